# baseline (speedup 1.0000x reference)
import jax
import jax.numpy as jnp
from jax.experimental import pallas as pl

B, N, CIN, FOV = 8, 1024, 3, 9
CH1, CH2 = 32, 32
ENC = 64
G1, G2 = 32, 32
K = 3
A = 5

P11 = 11
P = P11 * P11
PAD = P11 + 1
SHIFTS = tuple((dy - 1) * P11 + (dx - 1) for dy in range(3) for dx in range(3))

T = 128
PT = P * T
PADL = PAD * T


def _dot(a, b, out_dtype):
    return jax.lax.dot_general(a, b, (((1,), (0,)), ((), ())),
                               preferred_element_type=out_dtype)


def _encoder_kernel(xs_ref, ring_ref, w1_ref, b1_ref, w2_ref, b2_ref, we_ref,
                    be_ref, out_ref):
    xn = jnp.transpose(xs_ref[...]).astype(jnp.bfloat16)
    zc2 = jnp.zeros((1, 2 * T), jnp.bfloat16)
    zc12 = jnp.zeros((1, 12 * T), jnp.bfloat16)
    chans = []
    for c in range(CIN):
        rows = xn[c * FOV * FOV:(c + 1) * FOV * FOV, :]
        flat = rows.reshape(1, FOV * FOV * T)
        pieces = [zc12]
        for h in range(FOV):
            pieces.append(flat[:, h * FOV * T:(h + 1) * FOV * T])
            pieces.append(zc12 if h == FOV - 1 else zc2)
        chans.append(jnp.concatenate(pieces, axis=1))
    x = jnp.concatenate(chans + [ring_ref[...]], axis=0)
    xpad = jnp.pad(x, ((0, 0), (PADL, PADL)))
    p1 = jnp.concatenate(
        [xpad[:, (PAD + s) * T:(PAD + s) * T + PT] for s in SHIFTS], axis=0)
    y1 = jnp.maximum(_dot(w1_ref[...], p1, jnp.float32) + b1_ref[...],
                     0.0).astype(jnp.bfloat16)
    ypad = jnp.pad(y1, ((0, 0), (PADL, PADL)))
    p2 = jnp.concatenate(
        [ypad[:, (PAD + s) * T:(PAD + s) * T + PT] for s in SHIFTS], axis=0)
    y2 = jnp.maximum(_dot(w2_ref[...], p2, jnp.float32) + b2_ref[...],
                     0.0).astype(jnp.bfloat16)
    y2big = jnp.concatenate(
        [y2[:, p * T:(p + 1) * T] for p in range(P)], axis=0)
    e = jnp.maximum(_dot(we_ref[...], y2big, jnp.float32) + be_ref[...], 0.0)
    out_ref[...] = e


def _gnn_kernel(enc_ref, gso_ref, w1_ref, b1_ref, w2_ref, b2_ref,
                wa_ref, ba_ref, out_ref):
    S = gso_ref[0]
    xt = enc_ref[...]
    w1 = w1_ref[...]
    z1 = jax.lax.dot_general(S, xt, (((1,), (1,)), ((), ())),
                             preferred_element_type=jnp.float32)
    z2 = S @ z1
    x_w = jax.lax.dot_general(xt, w1[0], (((0,), (0,)), ((), ())),
                              preferred_element_type=jnp.float32)
    h = jnp.maximum(x_w + z1 @ w1[1] + z2 @ w1[2] + b1_ref[...], 0.0)
    w2 = w2_ref[...]
    u1 = S @ h
    u2 = S @ u1
    h2 = jnp.maximum(h @ w2[0] + u1 @ w2[1] + u2 @ w2[2] + b2_ref[...], 0.0)
    out_ref[0] = h2 @ wa_ref[...] + ba_ref[...]


@jax.jit
def kernel(states, gso, conv_w1, conv_b1, conv_w2, conv_b2, enc_w, enc_b,
           gnn_w1, gnn_b1, gnn_w2, gnn_b2, act_w, act_b):
    bn = B * N
    nb = bn // T
    xs = states.reshape(bn, CIN * FOV * FOV)
    ring = jnp.ones((P11, P11), jnp.bfloat16).at[1:1 + FOV, 1:1 + FOV].set(0)
    ring_lane = jnp.broadcast_to(ring.reshape(P, 1), (P, T)).reshape(1, PT)
    w1n = jnp.zeros((CH1, 9, CIN + 1), jnp.float32)
    w1n = w1n.at[:, :, :CIN].set(conv_w1.transpose(0, 2, 3, 1).reshape(CH1, 9, CIN))
    w1n = w1n.at[:, 4, CIN].set(-3e38)
    w1t = w1n.reshape(CH1, 9 * (CIN + 1)).astype(jnp.bfloat16)
    w2t = conv_w2.transpose(0, 2, 3, 1).reshape(CH2, 9 * CH1).astype(jnp.bfloat16)
    et = enc_w.reshape(CH2, FOV, FOV, ENC).transpose(1, 2, 0, 3)
    we = jnp.zeros((P11, P11, CH2, ENC), jnp.float32)
    we = we.at[1:1 + FOV, 1:1 + FOV].set(et)
    wet = we.reshape(P * CH2, ENC).T.astype(jnp.bfloat16)

    enc_t = pl.pallas_call(
        _encoder_kernel,
        grid=(nb,),
        in_specs=[
            pl.BlockSpec((T, CIN * FOV * FOV), lambda i: (i, 0)),
            pl.BlockSpec((1, PT), lambda i: (0, 0)),
            pl.BlockSpec((CH1, 9 * (CIN + 1)), lambda i: (0, 0)),
            pl.BlockSpec((CH1, 1), lambda i: (0, 0)),
            pl.BlockSpec((CH2, 9 * CH1), lambda i: (0, 0)),
            pl.BlockSpec((CH2, 1), lambda i: (0, 0)),
            pl.BlockSpec((ENC, P * CH2), lambda i: (0, 0)),
            pl.BlockSpec((ENC, 1), lambda i: (0, 0)),
        ],
        out_specs=pl.BlockSpec((ENC, T), lambda i: (0, i)),
        out_shape=jax.ShapeDtypeStruct((ENC, bn), jnp.float32),
    )(xs, ring_lane, w1t, conv_b1.reshape(CH1, 1).astype(jnp.bfloat16), w2t,
      conv_b2.reshape(CH2, 1).astype(jnp.bfloat16), wet, enc_b.reshape(ENC, 1))

    logits = pl.pallas_call(
        _gnn_kernel,
        grid=(B,),
        in_specs=[
            pl.BlockSpec((ENC, N), lambda b: (0, b)),
            pl.BlockSpec((1, N, N), lambda b: (b, 0, 0)),
            pl.BlockSpec((K, ENC, G1), lambda b: (0, 0, 0)),
            pl.BlockSpec((1, G1), lambda b: (0, 0)),
            pl.BlockSpec((K, G1, G2), lambda b: (0, 0, 0)),
            pl.BlockSpec((1, G2), lambda b: (0, 0)),
            pl.BlockSpec((G2, A), lambda b: (0, 0)),
            pl.BlockSpec((1, A), lambda b: (0, 0)),
        ],
        out_specs=pl.BlockSpec((1, N, A), lambda b: (b, 0, 0)),
        out_shape=jax.ShapeDtypeStruct((B, N, A), jnp.float32),
    )(enc_t, gso, gnn_w1, gnn_b1.reshape(1, G1),
      gnn_w2, gnn_b2.reshape(1, G2), act_w, act_b.reshape(1, A))

    return logits


# speedup vs baseline: 1.0777x; 1.0777x over previous
import jax
import jax.numpy as jnp
from jax.experimental import pallas as pl

B, N, CIN, FOV = 8, 1024, 3, 9
CH1, CH2 = 32, 32
ENC = 64
G1, G2 = 32, 32
K = 3
A = 5

P11 = 11
P = P11 * P11
PAD = P11 + 1
SHIFTS = tuple((dy - 1) * P11 + (dx - 1) for dy in range(3) for dx in range(3))

T = 128
PT = P * T
PADL = PAD * T


def _dot(a, b, out_dtype):
    return jax.lax.dot_general(a, b, (((1,), (0,)), ((), ())),
                               preferred_element_type=out_dtype)


def _encode_tile(x, w1, b1, w2, b2, we, be):
    xpad = jnp.pad(x, ((0, 0), (PADL, PADL)))
    p1 = jnp.concatenate(
        [xpad[:, (PAD + s) * T:(PAD + s) * T + PT] for s in SHIFTS], axis=0)
    y1 = jnp.maximum(_dot(w1, p1, jnp.float32) + b1,
                     0.0).astype(jnp.bfloat16)
    ypad = jnp.pad(y1, ((0, 0), (PADL, PADL)))
    p2 = jnp.concatenate(
        [ypad[:, (PAD + s) * T:(PAD + s) * T + PT] for s in SHIFTS], axis=0)
    y2 = jnp.maximum(_dot(w2, p2, jnp.float32) + b2,
                     0.0).astype(jnp.bfloat16)
    y2big = jnp.concatenate(
        [y2[:, p * T:(p + 1) * T] for p in range(P)], axis=0)
    return jnp.maximum(_dot(we, y2big, jnp.float32) + be, 0.0)


NTB = N // T


def _net_kernel(xt_ref, gso_ref, cw1_ref, cb1_ref, cw2_ref, cb2_ref, we_ref,
                be_ref, w1_ref, b1_ref, w2_ref, b2_ref, wa_ref, ba_ref,
                out_ref):
    cw1, cb1 = cw1_ref[...], cb1_ref[...]
    cw2, cb2 = cw2_ref[...], cb2_ref[...]
    we, be = we_ref[...], be_ref[...]
    xt = jnp.concatenate(
        [_encode_tile(xt_ref[0, t], cw1, cb1, cw2, cb2, we, be)
         for t in range(NTB)], axis=1)
    S = gso_ref[0]
    w1 = w1_ref[...]
    z1 = jax.lax.dot_general(S, xt, (((1,), (1,)), ((), ())),
                             preferred_element_type=jnp.float32)
    z2 = S @ z1
    x_w = jax.lax.dot_general(xt, w1[0], (((0,), (0,)), ((), ())),
                              preferred_element_type=jnp.float32)
    h = jnp.maximum(x_w + z1 @ w1[1] + z2 @ w1[2] + b1_ref[...], 0.0)
    w2 = w2_ref[...]
    u1 = S @ h
    u2 = S @ u1
    h2 = jnp.maximum(h @ w2[0] + u1 @ w2[1] + u2 @ w2[2] + b2_ref[...], 0.0)
    out_ref[0] = h2 @ wa_ref[...] + ba_ref[...]


@jax.jit
def kernel(states, gso, conv_w1, conv_b1, conv_w2, conv_b2, enc_w, enc_b,
           gnn_w1, gnn_b1, gnn_w2, gnn_b2, act_w, act_b):
    bn = B * N
    nb = bn // T
    r = states.astype(jnp.bfloat16)
    r = r.reshape(nb, T, CIN, FOV, FOV).transpose(0, 2, 3, 4, 1)
    xq = jnp.zeros((nb, CIN, P11, P11, T), jnp.bfloat16)
    xq = xq.at[:, :, 1:1 + FOV, 1:1 + FOV, :].set(r)
    ring = jnp.ones((P11, P11), jnp.bfloat16).at[1:1 + FOV, 1:1 + FOV].set(0)
    ring_b = jnp.broadcast_to(ring[None, None, :, :, None],
                              (nb, 1, P11, P11, T))
    xt = jnp.concatenate([xq, ring_b], axis=1).reshape(B, NTB, CIN + 1, PT)
    w1n = jnp.zeros((CH1, 9, CIN + 1), jnp.float32)
    w1n = w1n.at[:, :, :CIN].set(conv_w1.transpose(0, 2, 3, 1).reshape(CH1, 9, CIN))
    w1n = w1n.at[:, 4, CIN].set(-3e38)
    w1t = w1n.reshape(CH1, 9 * (CIN + 1)).astype(jnp.bfloat16)
    w2t = conv_w2.transpose(0, 2, 3, 1).reshape(CH2, 9 * CH1).astype(jnp.bfloat16)
    et = enc_w.reshape(CH2, FOV, FOV, ENC).transpose(1, 2, 0, 3)
    we = jnp.zeros((P11, P11, CH2, ENC), jnp.float32)
    we = we.at[1:1 + FOV, 1:1 + FOV].set(et)
    wet = we.reshape(P * CH2, ENC).T.astype(jnp.bfloat16)

    logits = pl.pallas_call(
        _net_kernel,
        grid=(B,),
        in_specs=[
            pl.BlockSpec((1, NTB, CIN + 1, PT), lambda b: (b, 0, 0, 0)),
            pl.BlockSpec((1, N, N), lambda b: (b, 0, 0)),
            pl.BlockSpec((CH1, 9 * (CIN + 1)), lambda b: (0, 0)),
            pl.BlockSpec((CH1, 1), lambda b: (0, 0)),
            pl.BlockSpec((CH2, 9 * CH1), lambda b: (0, 0)),
            pl.BlockSpec((CH2, 1), lambda b: (0, 0)),
            pl.BlockSpec((ENC, P * CH2), lambda b: (0, 0)),
            pl.BlockSpec((ENC, 1), lambda b: (0, 0)),
            pl.BlockSpec((K, ENC, G1), lambda b: (0, 0, 0)),
            pl.BlockSpec((1, G1), lambda b: (0, 0)),
            pl.BlockSpec((K, G1, G2), lambda b: (0, 0, 0)),
            pl.BlockSpec((1, G2), lambda b: (0, 0)),
            pl.BlockSpec((G2, A), lambda b: (0, 0)),
            pl.BlockSpec((1, A), lambda b: (0, 0)),
        ],
        out_specs=pl.BlockSpec((1, N, A), lambda b: (b, 0, 0)),
        out_shape=jax.ShapeDtypeStruct((B, N, A), jnp.float32),
    )(xt, gso, w1t, conv_b1.reshape(CH1, 1).astype(jnp.bfloat16), w2t,
      conv_b2.reshape(CH2, 1).astype(jnp.bfloat16), wet,
      enc_b.reshape(ENC, 1), gnn_w1, gnn_b1.reshape(1, G1),
      gnn_w2, gnn_b2.reshape(1, G2), act_w, act_b.reshape(1, A))

    return logits


# speedup vs baseline: 1.1353x; 1.0534x over previous
import jax
import jax.numpy as jnp
from jax.experimental import pallas as pl

B, N, CIN, FOV = 8, 1024, 3, 9
CH1, CH2 = 32, 32
ENC = 64
G1, G2 = 32, 32
K = 3
A = 5

P11 = 11
P = P11 * P11
PAD = P11 + 1
SHIFTS = tuple((dy - 1) * P11 + (dx - 1) for dy in range(3) for dx in range(3))

T = 128
PT = P * T
PADL = PAD * T


def _dot(a, b, out_dtype):
    return jax.lax.dot_general(a, b, (((1,), (0,)), ((), ())),
                               preferred_element_type=out_dtype)


def _build_tile(xs, ring):
    xn = jnp.transpose(xs).astype(jnp.bfloat16)
    zc2 = jnp.zeros((1, 2 * T), jnp.bfloat16)
    zc12 = jnp.zeros((1, 12 * T), jnp.bfloat16)
    chans = []
    for c in range(CIN):
        rows = xn[c * FOV * FOV:(c + 1) * FOV * FOV, :]
        flat = rows.reshape(1, FOV * FOV * T)
        pieces = [zc12]
        for h in range(FOV):
            pieces.append(flat[:, h * FOV * T:(h + 1) * FOV * T])
            pieces.append(zc12 if h == FOV - 1 else zc2)
        chans.append(jnp.concatenate(pieces, axis=1))
    return jnp.concatenate(chans + [ring], axis=0)


def _encode_tile(x, w1, b1, w2, b2, we, be):
    xpad = jnp.pad(x, ((0, 0), (PADL, PADL)))
    p1 = jnp.concatenate(
        [xpad[:, (PAD + s) * T:(PAD + s) * T + PT] for s in SHIFTS], axis=0)
    y1 = jnp.maximum(_dot(w1, p1, jnp.float32) + b1,
                     0.0).astype(jnp.bfloat16)
    ypad = jnp.pad(y1, ((0, 0), (PADL, PADL)))
    p2 = jnp.concatenate(
        [ypad[:, (PAD + s) * T:(PAD + s) * T + PT] for s in SHIFTS], axis=0)
    y2 = jnp.maximum(_dot(w2, p2, jnp.float32) + b2,
                     0.0).astype(jnp.bfloat16)
    y2big = jnp.concatenate(
        [y2[:, p * T:(p + 1) * T] for p in range(P)], axis=0)
    return jnp.maximum(_dot(we, y2big, jnp.float32) + be, 0.0)


NTB = N // T


def _net_kernel(xs_ref, ring_ref, gso_ref, cw1_ref, cb1_ref, cw2_ref,
                cb2_ref, we_ref, be_ref, w1_ref, b1_ref, w2_ref, b2_ref,
                wa_ref, ba_ref, out_ref):
    cw1, cb1 = cw1_ref[...], cb1_ref[...]
    cw2, cb2 = cw2_ref[...], cb2_ref[...]
    we, be = we_ref[...], be_ref[...]
    ring = ring_ref[...]
    xt = jnp.concatenate(
        [_encode_tile(_build_tile(xs_ref[0, t], ring),
                      cw1, cb1, cw2, cb2, we, be)
         for t in range(NTB)], axis=1)
    S = gso_ref[0]
    w1 = w1_ref[...]
    z1 = jax.lax.dot_general(S, xt, (((1,), (1,)), ((), ())),
                             preferred_element_type=jnp.float32)
    z2 = S @ z1
    x_w = jax.lax.dot_general(xt, w1[0], (((0,), (0,)), ((), ())),
                              preferred_element_type=jnp.float32)
    h = jnp.maximum(x_w + z1 @ w1[1] + z2 @ w1[2] + b1_ref[...], 0.0)
    w2 = w2_ref[...]
    u1 = S @ h
    u2 = S @ u1
    h2 = jnp.maximum(h @ w2[0] + u1 @ w2[1] + u2 @ w2[2] + b2_ref[...], 0.0)
    out_ref[0] = h2 @ wa_ref[...] + ba_ref[...]


@jax.jit
def kernel(states, gso, conv_w1, conv_b1, conv_w2, conv_b2, enc_w, enc_b,
           gnn_w1, gnn_b1, gnn_w2, gnn_b2, act_w, act_b):
    bn = B * N
    nb = bn // T
    xs = states.reshape(B, NTB, T, CIN * FOV * FOV)
    ring = jnp.ones((P11, P11), jnp.bfloat16).at[1:1 + FOV, 1:1 + FOV].set(0)
    ring_lane = jnp.broadcast_to(ring.reshape(P, 1), (P, T)).reshape(1, PT)
    w1n = jnp.zeros((CH1, 9, CIN + 1), jnp.float32)
    w1n = w1n.at[:, :, :CIN].set(conv_w1.transpose(0, 2, 3, 1).reshape(CH1, 9, CIN))
    w1n = w1n.at[:, 4, CIN].set(-3e38)
    w1t = w1n.reshape(CH1, 9 * (CIN + 1)).astype(jnp.bfloat16)
    w2t = conv_w2.transpose(0, 2, 3, 1).reshape(CH2, 9 * CH1).astype(jnp.bfloat16)
    et = enc_w.reshape(CH2, FOV, FOV, ENC).transpose(1, 2, 0, 3)
    we = jnp.zeros((P11, P11, CH2, ENC), jnp.float32)
    we = we.at[1:1 + FOV, 1:1 + FOV].set(et)
    wet = we.reshape(P * CH2, ENC).T.astype(jnp.bfloat16)

    logits = pl.pallas_call(
        _net_kernel,
        grid=(B,),
        in_specs=[
            pl.BlockSpec((1, NTB, T, CIN * FOV * FOV), lambda b: (b, 0, 0, 0)),
            pl.BlockSpec((1, PT), lambda b: (0, 0)),
            pl.BlockSpec((1, N, N), lambda b: (b, 0, 0)),
            pl.BlockSpec((CH1, 9 * (CIN + 1)), lambda b: (0, 0)),
            pl.BlockSpec((CH1, 1), lambda b: (0, 0)),
            pl.BlockSpec((CH2, 9 * CH1), lambda b: (0, 0)),
            pl.BlockSpec((CH2, 1), lambda b: (0, 0)),
            pl.BlockSpec((ENC, P * CH2), lambda b: (0, 0)),
            pl.BlockSpec((ENC, 1), lambda b: (0, 0)),
            pl.BlockSpec((K, ENC, G1), lambda b: (0, 0, 0)),
            pl.BlockSpec((1, G1), lambda b: (0, 0)),
            pl.BlockSpec((K, G1, G2), lambda b: (0, 0, 0)),
            pl.BlockSpec((1, G2), lambda b: (0, 0)),
            pl.BlockSpec((G2, A), lambda b: (0, 0)),
            pl.BlockSpec((1, A), lambda b: (0, 0)),
        ],
        out_specs=pl.BlockSpec((1, N, A), lambda b: (b, 0, 0)),
        out_shape=jax.ShapeDtypeStruct((B, N, A), jnp.float32),
    )(xs, ring_lane, gso, w1t, conv_b1.reshape(CH1, 1).astype(jnp.bfloat16), w2t,
      conv_b2.reshape(CH2, 1).astype(jnp.bfloat16), wet,
      enc_b.reshape(ENC, 1), gnn_w1, gnn_b1.reshape(1, G1),
      gnn_w2, gnn_b2.reshape(1, G2), act_w, act_b.reshape(1, A))

    return logits


# speedup vs baseline: 1.1854x; 1.0441x over previous
import jax
import jax.numpy as jnp
from jax.experimental import pallas as pl

B, N, CIN, FOV = 8, 1024, 3, 9
CH1, CH2 = 32, 32
ENC = 64
G1, G2 = 32, 32
K = 3
A = 5

P11 = 11
P = P11 * P11
PAD = P11 + 1
SHIFTS = tuple((dy - 1) * P11 + (dx - 1) for dy in range(3) for dx in range(3))

T = 128
PT = P * T
PADL = PAD * T


def _dot(a, b, out_dtype):
    return jax.lax.dot_general(a, b, (((1,), (0,)), ((), ())),
                               preferred_element_type=out_dtype)


def _build_tile(xs, ring):
    xn = jnp.transpose(xs)
    zc2 = jnp.zeros((1, 2 * T), jnp.bfloat16)
    zc12 = jnp.zeros((1, 12 * T), jnp.bfloat16)
    chans = []
    for c in range(CIN):
        rows = xn[c * FOV * FOV:(c + 1) * FOV * FOV, :]
        flat = rows.reshape(1, FOV * FOV * T)
        pieces = [zc12]
        for h in range(FOV):
            pieces.append(flat[:, h * FOV * T:(h + 1) * FOV * T])
            pieces.append(zc12 if h == FOV - 1 else zc2)
        chans.append(jnp.concatenate(pieces, axis=1))
    return jnp.concatenate(chans + [ring], axis=0)


def _encode_tile(x, w1, b1, w2, b2, we, be):
    xpad = jnp.pad(x, ((0, 0), (PADL, PADL)))
    p1 = jnp.concatenate(
        [xpad[:, (PAD + s) * T:(PAD + s) * T + PT] for s in SHIFTS], axis=0)
    y1 = jnp.maximum(_dot(w1, p1, jnp.float32) + b1,
                     0.0).astype(jnp.bfloat16)
    ypad = jnp.pad(y1, ((0, 0), (PADL, PADL)))
    p2 = jnp.concatenate(
        [ypad[:, (PAD + s) * T:(PAD + s) * T + PT] for s in SHIFTS], axis=0)
    y2 = jnp.maximum(_dot(w2, p2, jnp.float32) + b2,
                     0.0).astype(jnp.bfloat16)
    y2big = jnp.concatenate(
        [y2[:, p * T:(p + 1) * T] for p in range(P)], axis=0)
    return jnp.maximum(_dot(we, y2big, jnp.float32) + be, 0.0)


NTB = N // T


def _net_kernel(xs_ref, ring_ref, gso_ref, cw1_ref, cb1_ref, cw2_ref,
                cb2_ref, we_ref, be_ref, w1_ref, b1_ref, w2_ref, b2_ref,
                wa_ref, ba_ref, out_ref):
    cw1, cb1 = cw1_ref[...], cb1_ref[...]
    cw2, cb2 = cw2_ref[...], cb2_ref[...]
    we, be = we_ref[...], be_ref[...]
    ring = ring_ref[...]
    xt = jnp.concatenate(
        [_encode_tile(_build_tile(xs_ref[0, t], ring),
                      cw1, cb1, cw2, cb2, we, be)
         for t in range(NTB)], axis=1)
    S = gso_ref[0]
    w1 = w1_ref[...]
    z1 = jax.lax.dot_general(S, xt, (((1,), (1,)), ((), ())),
                             preferred_element_type=jnp.float32)
    z2 = S @ z1
    x_w = jax.lax.dot_general(xt, w1[0], (((0,), (0,)), ((), ())),
                              preferred_element_type=jnp.float32)
    h = jnp.maximum(x_w + z1 @ w1[1] + z2 @ w1[2] + b1_ref[...], 0.0)
    w2 = w2_ref[...]
    u1 = S @ h
    u2 = S @ u1
    h2 = jnp.maximum(h @ w2[0] + u1 @ w2[1] + u2 @ w2[2] + b2_ref[...], 0.0)
    out_ref[0] = h2 @ wa_ref[...] + ba_ref[...]


@jax.jit
def kernel(states, gso, conv_w1, conv_b1, conv_w2, conv_b2, enc_w, enc_b,
           gnn_w1, gnn_b1, gnn_w2, gnn_b2, act_w, act_b):
    bn = B * N
    nb = bn // T
    xs = states.astype(jnp.bfloat16).reshape(B, NTB, T, CIN * FOV * FOV)
    ring = jnp.ones((P11, P11), jnp.bfloat16).at[1:1 + FOV, 1:1 + FOV].set(0)
    ring_lane = jnp.broadcast_to(ring.reshape(P, 1), (P, T)).reshape(1, PT)
    w1n = jnp.zeros((CH1, 9, CIN + 1), jnp.float32)
    w1n = w1n.at[:, :, :CIN].set(conv_w1.transpose(0, 2, 3, 1).reshape(CH1, 9, CIN))
    w1n = w1n.at[:, 4, CIN].set(-3e38)
    w1t = w1n.reshape(CH1, 9 * (CIN + 1)).astype(jnp.bfloat16)
    w2t = conv_w2.transpose(0, 2, 3, 1).reshape(CH2, 9 * CH1).astype(jnp.bfloat16)
    et = enc_w.reshape(CH2, FOV, FOV, ENC).transpose(1, 2, 0, 3)
    we = jnp.zeros((P11, P11, CH2, ENC), jnp.float32)
    we = we.at[1:1 + FOV, 1:1 + FOV].set(et)
    wet = we.reshape(P * CH2, ENC).T.astype(jnp.bfloat16)

    logits = pl.pallas_call(
        _net_kernel,
        grid=(B,),
        in_specs=[
            pl.BlockSpec((1, NTB, T, CIN * FOV * FOV), lambda b: (b, 0, 0, 0)),
            pl.BlockSpec((1, PT), lambda b: (0, 0)),
            pl.BlockSpec((1, N, N), lambda b: (b, 0, 0)),
            pl.BlockSpec((CH1, 9 * (CIN + 1)), lambda b: (0, 0)),
            pl.BlockSpec((CH1, 1), lambda b: (0, 0)),
            pl.BlockSpec((CH2, 9 * CH1), lambda b: (0, 0)),
            pl.BlockSpec((CH2, 1), lambda b: (0, 0)),
            pl.BlockSpec((ENC, P * CH2), lambda b: (0, 0)),
            pl.BlockSpec((ENC, 1), lambda b: (0, 0)),
            pl.BlockSpec((K, ENC, G1), lambda b: (0, 0, 0)),
            pl.BlockSpec((1, G1), lambda b: (0, 0)),
            pl.BlockSpec((K, G1, G2), lambda b: (0, 0, 0)),
            pl.BlockSpec((1, G2), lambda b: (0, 0)),
            pl.BlockSpec((G2, A), lambda b: (0, 0)),
            pl.BlockSpec((1, A), lambda b: (0, 0)),
        ],
        out_specs=pl.BlockSpec((1, N, A), lambda b: (b, 0, 0)),
        out_shape=jax.ShapeDtypeStruct((B, N, A), jnp.float32),
    )(xs, ring_lane, gso, w1t, conv_b1.reshape(CH1, 1).astype(jnp.bfloat16), w2t,
      conv_b2.reshape(CH2, 1).astype(jnp.bfloat16), wet,
      enc_b.reshape(ENC, 1), gnn_w1, gnn_b1.reshape(1, G1),
      gnn_w2, gnn_b2.reshape(1, G2), act_w, act_b.reshape(1, A))

    return logits
